# Initial kernel scaffold; baseline (speedup 1.0000x reference)
import jax
import jax.numpy as jnp
from jax.experimental import pallas as pl


def kernel(ids, negative_samples, confidence, min_embedding, delta_embedding, rel_trans_for_head, rel_scale_for_head, rel_trans_for_tail, rel_scale_for_tail):
    raise NotImplementedError("write your pallas kernel here")


# speedup vs baseline: 2.4769x; 2.4769x over previous
import functools

import jax
import jax.numpy as jnp
from jax import lax
from jax.experimental import pallas as pl
from jax.experimental.pallas import tpu as pltpu
from jax.experimental.pallas import tpu_sc as plsc

N_ENT = 100000
N_REL = 1000
DIM = 128
B = 16384
BETA = 1.0
EPS = 1e-23
REG_DELTA = 0.05
REG_MIN = 0.0005
REG_REL = 0.0005

_NC = 2
_NS = 16
_NW = _NC * _NS
_BPW = B // _NW
_CHUNK = 128
_NCHUNK = _BPW // _CHUNK


def _sc_gather_body(min_e, delta_e, rth_t, rsh_t, rtt_t, rst_t,
                    h_i, t_i, nh_i, nt_i, r_i,
                    o_mh, o_dh, o_mt, o_dt,
                    o_mnh, o_dnh, o_mnt, o_dnt,
                    o_rth, o_rsh, o_rtt, o_rst,
                    idx_v, rows_v, sem):
    wid = lax.axis_index("s") * _NC + lax.axis_index("c")
    base = wid * _BPW
    groups = [
        (h_i, ((min_e, o_mh), (delta_e, o_dh))),
        (t_i, ((min_e, o_mt), (delta_e, o_dt))),
        (nh_i, ((min_e, o_mnh), (delta_e, o_dnh))),
        (nt_i, ((min_e, o_mnt), (delta_e, o_dnt))),
        (r_i, ((rth_t, o_rth), (rsh_t, o_rsh), (rtt_t, o_rtt), (rst_t, o_rst))),
    ]
    for c in range(_NCHUNK):
        row0 = base + c * _CHUNK
        for idx_hbm, pairs in groups:
            pltpu.sync_copy(idx_hbm.at[pl.ds(row0, _CHUNK)], idx_v)
            for table, out in pairs:
                pltpu.async_copy(table.at[idx_v], rows_v, sem).wait()
                pltpu.sync_copy(rows_v, out.at[pl.ds(row0, _CHUNK)])


@jax.jit
def _sc_gather(min_e, delta_e, rth_t, rsh_t, rtt_t, rst_t,
               h, t, nh, nt, r):
    row = jax.ShapeDtypeStruct((B, DIM), jnp.float32)
    fn = pl.kernel(
        _sc_gather_body,
        out_type=[row] * 12,
        mesh=plsc.VectorSubcoreMesh(core_axis_name="c", subcore_axis_name="s"),
        scratch_types=[
            pltpu.VMEM((_CHUNK,), jnp.int32),
            pltpu.VMEM((_CHUNK, DIM), jnp.float32),
            pltpu.SemaphoreType.DMA,
        ],
    )
    return fn(min_e, delta_e, rth_t, rsh_t, rtt_t, rst_t, h, t, nh, nt, r)


def _log1p(x):
    u = 1.0 + x
    d = u - 1.0
    return jnp.where(d == 0.0, x, jnp.log(u) * (x / d))


def _logaddexp(a, b):
    mx = jnp.maximum(a, b)
    return mx + _log1p(jnp.exp(-jnp.abs(a - b)))


def _softplus(x):
    return jnp.maximum(x, 0.0) + _log1p(jnp.exp(-jnp.abs(x)))


def _log_volume(bmin, bmax):
    return jnp.sum(jnp.log(_softplus((bmax - bmin) / BETA) * BETA + EPS),
                   axis=1, keepdims=True)


def _pred(h_min, h_max, t_min, t_max):
    meet_min = BETA * _logaddexp(h_min / BETA, t_min / BETA)
    meet_max = -BETA * _logaddexp(-h_max / BETA, -t_max / BETA)
    log_int = _log_volume(meet_min, meet_max)
    log_tail = _log_volume(t_min, t_max)
    return jnp.exp(jnp.minimum(log_int - log_tail, 0.0))


def _rownorm(x):
    return jnp.sqrt(jnp.sum(x * x, axis=1, keepdims=True))


_BB = 512
_NB = B // _BB


def _tc_loss_body(mh, dh, mt, dt, mnh, dnh, mnt, dnt,
                  rth, rsh, rtt, rst, conf, out_ref, acc_ref):
    i = pl.program_id(0)

    @pl.when(i == 0)
    def _():
        acc_ref[0] = 0.0

    sc_h = jnp.exp(rsh[...])
    sc_t = jnp.exp(rst[...])
    edh = jnp.exp(dh[...])
    edt = jnp.exp(dt[...])

    h_min = mh[...] * sc_h + rth[...]
    h_max = h_min + edh * sc_h
    t_min = mt[...] * sc_t + rtt[...]
    t_max = t_min + edt * sc_t
    pos = _pred(h_min, h_max, t_min, t_max)

    nh_min = mnh[...] * sc_h + rth[...]
    nh_max = nh_min + jnp.exp(dnh[...]) * sc_h
    nt_min = mnt[...] * sc_t + rtt[...]
    nt_max = nt_min + jnp.exp(dnt[...]) * sc_t
    neg = _pred(nh_min, nh_max, nt_min, nt_max)

    se = (pos - conf[...]) ** 2 + neg * neg
    reg = (REG_DELTA * (_rownorm(edh) + _rownorm(edt))
           + REG_MIN * (_rownorm(mh[...]) + _rownorm(mt[...]))
           + REG_REL * (_rownorm(jnp.exp(rth[...])) + _rownorm(jnp.exp(rtt[...])))
           + REG_REL * (_rownorm(sc_h) + _rownorm(sc_t)))
    acc_ref[0] += jnp.sum(se) + jnp.sum(reg)

    @pl.when(i == _NB - 1)
    def _():
        out_ref[...] = jnp.full((1, 1), acc_ref[0] * (1.0 / B), jnp.float32)


@jax.jit
def _tc_loss(gathered, conf2d):
    row_spec = pl.BlockSpec((_BB, DIM), lambda i: (i, 0))
    conf_spec = pl.BlockSpec((_BB, 1), lambda i: (i, 0))
    return pl.pallas_call(
        _tc_loss_body,
        grid=(_NB,),
        in_specs=[row_spec] * 12 + [conf_spec],
        out_specs=pl.BlockSpec((1, 1), lambda i: (0, 0)),
        out_shape=jax.ShapeDtypeStruct((1, 1), jnp.float32),
        scratch_shapes=[pltpu.SMEM((1,), jnp.float32)],
    )(*gathered, conf2d)


def kernel(ids, negative_samples, confidence, min_embedding, delta_embedding,
           rel_trans_for_head, rel_scale_for_head, rel_trans_for_tail,
           rel_scale_for_tail):
    ids = ids.astype(jnp.int32)
    neg = negative_samples.astype(jnp.int32)
    h = ids[:, 0]
    r = ids[:, 1]
    t = ids[:, 2]
    nh = neg[:, 0]
    nt = neg[:, 2]
    gathered = _sc_gather(min_embedding, delta_embedding,
                          rel_trans_for_head, rel_scale_for_head,
                          rel_trans_for_tail, rel_scale_for_tail,
                          h, t, nh, nt, r)
    loss = _tc_loss(gathered, confidence.reshape(B, 1))
    return loss.reshape(())


# speedup vs baseline: 3.1093x; 1.2553x over previous
import functools

import jax
import jax.numpy as jnp
from jax import lax
from jax.experimental import pallas as pl
from jax.experimental.pallas import tpu as pltpu
from jax.experimental.pallas import tpu_sc as plsc

N_ENT = 100000
N_REL = 1000
DIM = 128
B = 16384
BETA = 1.0
EPS = 1e-23
REG_DELTA = 0.05
REG_MIN = 0.0005
REG_REL = 0.0005

_NC = 2
_NS = 16
_NW = _NC * _NS
_BPW = B // _NW
_CHUNK = 128
_NCHUNK = _BPW // _CHUNK


def _sc_gather_body(min_e, delta_e, rth_t, rsh_t, rtt_t, rst_t,
                    h_i, t_i, nh_i, nt_i, r_i,
                    o_mh, o_dh, o_mt, o_dt,
                    o_mnh, o_dnh, o_mnt, o_dnt,
                    o_rth, o_rsh, o_rtt, o_rst,
                    idx_all, bufs, isem, gsems, ssems):
    wid = lax.axis_index("s") * _NC + lax.axis_index("c")
    base = wid * _BPW
    groups = [
        (h_i, ((min_e, o_mh), (delta_e, o_dh))),
        (t_i, ((min_e, o_mt), (delta_e, o_dt))),
        (nh_i, ((min_e, o_mnh), (delta_e, o_dnh))),
        (nt_i, ((min_e, o_mnt), (delta_e, o_dnt))),
        (r_i, ((rth_t, o_rth), (rsh_t, o_rsh), (rtt_t, o_rtt), (rst_t, o_rst))),
    ]
    idx_copies = []
    for g, (idx_hbm, _) in enumerate(groups):
        for c in range(_NCHUNK):
            idx_copies.append(pltpu.async_copy(
                idx_hbm.at[pl.ds(base + c * _CHUNK, _CHUNK)],
                idx_all.at[g * _NCHUNK + c], isem))
    for cp in idx_copies:
        cp.wait()

    units = []
    for c in range(_NCHUNK):
        for g, (_, pairs) in enumerate(groups):
            for table, out in pairs:
                units.append((g * _NCHUNK + c, table, out, base + c * _CHUNK))

    nbuf = len(gsems)
    gathers = [None] * nbuf
    stores = [None] * nbuf

    def start_gather(k):
        slot = k % nbuf
        j, table, _, _ = units[k]
        if stores[slot] is not None:
            stores[slot].wait()
        gathers[slot] = pltpu.async_copy(
            table.at[idx_all.at[j]], bufs.at[slot], gsems[slot])

    start_gather(0)
    for k in range(len(units)):
        if k + 1 < len(units):
            start_gather(k + 1)
        slot = k % nbuf
        _, _, out, row0 = units[k]
        gathers[slot].wait()
        stores[slot] = pltpu.async_copy(
            bufs.at[slot], out.at[pl.ds(row0, _CHUNK)], ssems[slot])
    for st in stores:
        if st is not None:
            st.wait()


_NBUF = 4


@jax.jit
def _sc_gather(min_e, delta_e, rth_t, rsh_t, rtt_t, rst_t,
               h, t, nh, nt, r):
    row = jax.ShapeDtypeStruct((B, DIM), jnp.float32)
    fn = pl.kernel(
        _sc_gather_body,
        out_type=[row] * 12,
        mesh=plsc.VectorSubcoreMesh(core_axis_name="c", subcore_axis_name="s"),
        scratch_types=[
            pltpu.VMEM((5 * _NCHUNK, _CHUNK), jnp.int32),
            pltpu.VMEM((_NBUF, _CHUNK, DIM), jnp.float32),
            pltpu.SemaphoreType.DMA,
            [pltpu.SemaphoreType.DMA] * _NBUF,
            [pltpu.SemaphoreType.DMA] * _NBUF,
        ],
    )
    return fn(min_e, delta_e, rth_t, rsh_t, rtt_t, rst_t, h, t, nh, nt, r)


def _log1p(x):
    u = 1.0 + x
    d = u - 1.0
    return jnp.where(d == 0.0, x, jnp.log(u) * (x / d))


def _logaddexp(a, b):
    mx = jnp.maximum(a, b)
    return mx + _log1p(jnp.exp(-jnp.abs(a - b)))


def _softplus(x):
    return jnp.maximum(x, 0.0) + _log1p(jnp.exp(-jnp.abs(x)))


def _log_volume(bmin, bmax):
    return jnp.sum(jnp.log(_softplus((bmax - bmin) / BETA) * BETA + EPS),
                   axis=1, keepdims=True)


def _pred(h_min, h_max, t_min, t_max):
    meet_min = BETA * _logaddexp(h_min / BETA, t_min / BETA)
    meet_max = -BETA * _logaddexp(-h_max / BETA, -t_max / BETA)
    log_int = _log_volume(meet_min, meet_max)
    log_tail = _log_volume(t_min, t_max)
    return jnp.exp(jnp.minimum(log_int - log_tail, 0.0))


def _rownorm(x):
    return jnp.sqrt(jnp.sum(x * x, axis=1, keepdims=True))


_BB = 512
_NB = B // _BB


def _tc_loss_body(mh, dh, mt, dt, mnh, dnh, mnt, dnt,
                  rth, rsh, rtt, rst, conf, out_ref, acc_ref):
    i = pl.program_id(0)

    @pl.when(i == 0)
    def _():
        acc_ref[0] = 0.0

    sc_h = jnp.exp(rsh[...])
    sc_t = jnp.exp(rst[...])
    edh = jnp.exp(dh[...])
    edt = jnp.exp(dt[...])

    h_min = mh[...] * sc_h + rth[...]
    h_max = h_min + edh * sc_h
    t_min = mt[...] * sc_t + rtt[...]
    t_max = t_min + edt * sc_t
    pos = _pred(h_min, h_max, t_min, t_max)

    nh_min = mnh[...] * sc_h + rth[...]
    nh_max = nh_min + jnp.exp(dnh[...]) * sc_h
    nt_min = mnt[...] * sc_t + rtt[...]
    nt_max = nt_min + jnp.exp(dnt[...]) * sc_t
    neg = _pred(nh_min, nh_max, nt_min, nt_max)

    se = (pos - conf[...]) ** 2 + neg * neg
    reg = (REG_DELTA * (_rownorm(edh) + _rownorm(edt))
           + REG_MIN * (_rownorm(mh[...]) + _rownorm(mt[...]))
           + REG_REL * (_rownorm(jnp.exp(rth[...])) + _rownorm(jnp.exp(rtt[...])))
           + REG_REL * (_rownorm(sc_h) + _rownorm(sc_t)))
    acc_ref[0] += jnp.sum(se) + jnp.sum(reg)

    @pl.when(i == _NB - 1)
    def _():
        out_ref[...] = jnp.full((1, 1), acc_ref[0] * (1.0 / B), jnp.float32)


@jax.jit
def _tc_loss(gathered, conf2d):
    row_spec = pl.BlockSpec((_BB, DIM), lambda i: (i, 0))
    conf_spec = pl.BlockSpec((_BB, 1), lambda i: (i, 0))
    return pl.pallas_call(
        _tc_loss_body,
        grid=(_NB,),
        in_specs=[row_spec] * 12 + [conf_spec],
        out_specs=pl.BlockSpec((1, 1), lambda i: (0, 0)),
        out_shape=jax.ShapeDtypeStruct((1, 1), jnp.float32),
        scratch_shapes=[pltpu.SMEM((1,), jnp.float32)],
    )(*gathered, conf2d)


def kernel(ids, negative_samples, confidence, min_embedding, delta_embedding,
           rel_trans_for_head, rel_scale_for_head, rel_trans_for_tail,
           rel_scale_for_tail):
    ids = ids.astype(jnp.int32)
    neg = negative_samples.astype(jnp.int32)
    h = ids[:, 0]
    r = ids[:, 1]
    t = ids[:, 2]
    nh = neg[:, 0]
    nt = neg[:, 2]
    gathered = _sc_gather(min_embedding, delta_embedding,
                          rel_trans_for_head, rel_scale_for_head,
                          rel_trans_for_tail, rel_scale_for_tail,
                          h, t, nh, nt, r)
    loss = _tc_loss(gathered, confidence.reshape(B, 1))
    return loss.reshape(())


# speedup vs baseline: 3.3132x; 1.0656x over previous
import functools

import jax
import jax.numpy as jnp
from jax import lax
from jax.experimental import pallas as pl
from jax.experimental.pallas import tpu as pltpu
from jax.experimental.pallas import tpu_sc as plsc

N_ENT = 100000
N_REL = 1000
DIM = 128
B = 16384
BETA = 1.0
EPS = 1e-23
REG_DELTA = 0.05
REG_MIN = 0.0005
REG_REL = 0.0005

_NC = 2
_NS = 16
_NW = _NC * _NS
_CHUNK = 128
_NSPLIT = 2


def _sc_gather_body(nrows,
                    min_e, delta_e, rth_t, rsh_t, rtt_t, rst_t,
                    h_i, t_i, nh_i, nt_i, r_i,
                    o_mh, o_dh, o_mt, o_dt,
                    o_mnh, o_dnh, o_mnt, o_dnt,
                    o_rth, o_rsh, o_rtt, o_rst,
                    idx_all, bufs, isem, gsems, ssems):
    _NCHUNK = nrows // _NW // _CHUNK
    wid = lax.axis_index("s") * _NC + lax.axis_index("c")
    base = wid * (nrows // _NW)
    groups = [
        (h_i, ((min_e, o_mh), (delta_e, o_dh))),
        (t_i, ((min_e, o_mt), (delta_e, o_dt))),
        (nh_i, ((min_e, o_mnh), (delta_e, o_dnh))),
        (nt_i, ((min_e, o_mnt), (delta_e, o_dnt))),
        (r_i, ((rth_t, o_rth), (rsh_t, o_rsh), (rtt_t, o_rtt), (rst_t, o_rst))),
    ]
    idx_copies = []
    for g, (idx_hbm, _) in enumerate(groups):
        for c in range(_NCHUNK):
            idx_copies.append(pltpu.async_copy(
                idx_hbm.at[pl.ds(base + c * _CHUNK, _CHUNK)],
                idx_all.at[g * _NCHUNK + c], isem))
    for cp in idx_copies:
        cp.wait()

    units = []
    for c in range(_NCHUNK):
        for g, (_, pairs) in enumerate(groups):
            for table, out in pairs:
                units.append((g * _NCHUNK + c, table, out, base + c * _CHUNK))

    nbuf = len(gsems)
    gathers = [None] * nbuf
    stores = [None] * nbuf

    def start_gather(k):
        slot = k % nbuf
        j, table, _, _ = units[k]
        if stores[slot] is not None:
            stores[slot].wait()
        gathers[slot] = pltpu.async_copy(
            table.at[idx_all.at[j]], bufs.at[slot], gsems[slot])

    start_gather(0)
    for k in range(len(units)):
        if k + 1 < len(units):
            start_gather(k + 1)
        slot = k % nbuf
        _, _, out, row0 = units[k]
        gathers[slot].wait()
        stores[slot] = pltpu.async_copy(
            bufs.at[slot], out.at[pl.ds(row0, _CHUNK)], ssems[slot])
    for st in stores:
        if st is not None:
            st.wait()


_NBUF = 4


def _make_sc_gather(nrows):
    row = jax.ShapeDtypeStruct((nrows, DIM), jnp.float32)
    nchunk = nrows // _NW // _CHUNK
    return pl.kernel(
        functools.partial(_sc_gather_body, nrows),
        out_type=[row] * 12,
        mesh=plsc.VectorSubcoreMesh(core_axis_name="c", subcore_axis_name="s"),
        scratch_types=[
            pltpu.VMEM((5 * nchunk, _CHUNK), jnp.int32),
            pltpu.VMEM((_NBUF, _CHUNK, DIM), jnp.float32),
            pltpu.SemaphoreType.DMA,
            [pltpu.SemaphoreType.DMA] * _NBUF,
            [pltpu.SemaphoreType.DMA] * _NBUF,
        ],
    )


def _log1p(x):
    u = 1.0 + x
    d = u - 1.0
    return jnp.where(d == 0.0, x, jnp.log(u) * (x / d))


def _logaddexp(a, b):
    mx = jnp.maximum(a, b)
    return mx + _log1p(jnp.exp(-jnp.abs(a - b)))


def _softplus(x):
    return jnp.maximum(x, 0.0) + _log1p(jnp.exp(-jnp.abs(x)))


def _log_volume(bmin, bmax):
    return jnp.sum(jnp.log(_softplus((bmax - bmin) / BETA) * BETA + EPS),
                   axis=1, keepdims=True)


def _pred(h_min, h_max, t_min, t_max):
    meet_min = BETA * _logaddexp(h_min / BETA, t_min / BETA)
    meet_max = -BETA * _logaddexp(-h_max / BETA, -t_max / BETA)
    log_int = _log_volume(meet_min, meet_max)
    log_tail = _log_volume(t_min, t_max)
    return jnp.exp(jnp.minimum(log_int - log_tail, 0.0))


def _rownorm(x):
    return jnp.sqrt(jnp.sum(x * x, axis=1, keepdims=True))


_BB = 512


def _tc_loss_body(nb, mh, dh, mt, dt, mnh, dnh, mnt, dnt,
                  rth, rsh, rtt, rst, conf, out_ref, acc_ref):
    i = pl.program_id(0)

    @pl.when(i == 0)
    def _():
        acc_ref[0] = 0.0

    sc_h = jnp.exp(rsh[...])
    sc_t = jnp.exp(rst[...])
    edh = jnp.exp(dh[...])
    edt = jnp.exp(dt[...])

    h_min = mh[...] * sc_h + rth[...]
    h_max = h_min + edh * sc_h
    t_min = mt[...] * sc_t + rtt[...]
    t_max = t_min + edt * sc_t
    pos = _pred(h_min, h_max, t_min, t_max)

    nh_min = mnh[...] * sc_h + rth[...]
    nh_max = nh_min + jnp.exp(dnh[...]) * sc_h
    nt_min = mnt[...] * sc_t + rtt[...]
    nt_max = nt_min + jnp.exp(dnt[...]) * sc_t
    neg = _pred(nh_min, nh_max, nt_min, nt_max)

    se = (pos - conf[...]) ** 2 + neg * neg
    reg = (REG_DELTA * (_rownorm(edh) + _rownorm(edt))
           + REG_MIN * (_rownorm(mh[...]) + _rownorm(mt[...]))
           + REG_REL * (_rownorm(jnp.exp(rth[...])) + _rownorm(jnp.exp(rtt[...])))
           + REG_REL * (_rownorm(sc_h) + _rownorm(sc_t)))
    acc_ref[0] += jnp.sum(se) + jnp.sum(reg)

    @pl.when(i == nb - 1)
    def _():
        out_ref[...] = jnp.full((1, 1), acc_ref[0], jnp.float32)


def _make_tc_loss(nrows):
    nb = nrows // _BB
    row_spec = pl.BlockSpec((_BB, DIM), lambda i: (i, 0))
    conf_spec = pl.BlockSpec((_BB, 1), lambda i: (i, 0))
    return pl.pallas_call(
        functools.partial(_tc_loss_body, nb),
        grid=(nb,),
        in_specs=[row_spec] * 12 + [conf_spec],
        out_specs=pl.BlockSpec((1, 1), lambda i: (0, 0)),
        out_shape=jax.ShapeDtypeStruct((1, 1), jnp.float32),
        scratch_shapes=[pltpu.SMEM((1,), jnp.float32)],
    )


def kernel(ids, negative_samples, confidence, min_embedding, delta_embedding,
           rel_trans_for_head, rel_scale_for_head, rel_trans_for_tail,
           rel_scale_for_tail):
    ids = ids.astype(jnp.int32)
    neg = negative_samples.astype(jnp.int32)
    h = ids[:, 0]
    r = ids[:, 1]
    t = ids[:, 2]
    nh = neg[:, 0]
    nt = neg[:, 2]
    n = B // _NSPLIT
    sc_fn = _make_sc_gather(n)
    tc_fn = _make_tc_loss(n)
    conf2d = confidence.reshape(B, 1)
    partials = []
    for s in range(_NSPLIT):
        sl = slice(s * n, (s + 1) * n)
        gathered = sc_fn(min_embedding, delta_embedding,
                         rel_trans_for_head, rel_scale_for_head,
                         rel_trans_for_tail, rel_scale_for_tail,
                         h[sl], t[sl], nh[sl], nt[sl], r[sl])
        partials.append(tc_fn(*gathered, conf2d[sl]))
    total = partials[0]
    for p in partials[1:]:
        total = total + p
    return (total * (1.0 / B)).reshape(())


# speedup vs baseline: 3.4522x; 1.0420x over previous
import functools

import jax
import jax.numpy as jnp
from jax import lax
from jax.experimental import pallas as pl
from jax.experimental.pallas import tpu as pltpu
from jax.experimental.pallas import tpu_sc as plsc

N_ENT = 100000
N_REL = 1000
DIM = 128
B = 16384
BETA = 1.0
EPS = 1e-23
REG_DELTA = 0.05
REG_MIN = 0.0005
REG_REL = 0.0005

_NC = 2
_NS = 16
_NW = _NC * _NS
_CHUNK = 128
_NSPLIT = 4


def _sc_gather_body(nrows,
                    min_e, delta_e, rth_t, rsh_t, rtt_t, rst_t,
                    h_i, t_i, nh_i, nt_i, r_i,
                    o_mh, o_dh, o_mt, o_dt,
                    o_mnh, o_dnh, o_mnt, o_dnt,
                    o_rth, o_rsh, o_rtt, o_rst,
                    idx_all, bufs, isem, gsems, ssems):
    _NCHUNK = nrows // _NW // _CHUNK
    wid = lax.axis_index("s") * _NC + lax.axis_index("c")
    base = wid * (nrows // _NW)
    groups = [
        (h_i, ((min_e, o_mh), (delta_e, o_dh))),
        (t_i, ((min_e, o_mt), (delta_e, o_dt))),
        (nh_i, ((min_e, o_mnh), (delta_e, o_dnh))),
        (nt_i, ((min_e, o_mnt), (delta_e, o_dnt))),
        (r_i, ((rth_t, o_rth), (rsh_t, o_rsh), (rtt_t, o_rtt), (rst_t, o_rst))),
    ]
    idx_copies = []
    for g, (idx_hbm, _) in enumerate(groups):
        for c in range(_NCHUNK):
            idx_copies.append(pltpu.async_copy(
                idx_hbm.at[pl.ds(base + c * _CHUNK, _CHUNK)],
                idx_all.at[g * _NCHUNK + c], isem))
    for cp in idx_copies:
        cp.wait()

    units = []
    for c in range(_NCHUNK):
        for g, (_, pairs) in enumerate(groups):
            for table, out in pairs:
                units.append((g * _NCHUNK + c, table, out, base + c * _CHUNK))

    nbuf = len(gsems)
    gathers = [None] * nbuf
    stores = [None] * nbuf

    def start_gather(k):
        slot = k % nbuf
        j, table, _, _ = units[k]
        if stores[slot] is not None:
            stores[slot].wait()
        gathers[slot] = pltpu.async_copy(
            table.at[idx_all.at[j]], bufs.at[slot], gsems[slot])

    start_gather(0)
    for k in range(len(units)):
        if k + 1 < len(units):
            start_gather(k + 1)
        slot = k % nbuf
        _, _, out, row0 = units[k]
        gathers[slot].wait()
        stores[slot] = pltpu.async_copy(
            bufs.at[slot], out.at[pl.ds(row0, _CHUNK)], ssems[slot])
    for st in stores:
        if st is not None:
            st.wait()


_NBUF = 4


def _make_sc_gather(nrows):
    row = jax.ShapeDtypeStruct((nrows, DIM), jnp.float32)
    nchunk = nrows // _NW // _CHUNK
    return pl.kernel(
        functools.partial(_sc_gather_body, nrows),
        out_type=[row] * 12,
        mesh=plsc.VectorSubcoreMesh(core_axis_name="c", subcore_axis_name="s"),
        scratch_types=[
            pltpu.VMEM((5 * nchunk, _CHUNK), jnp.int32),
            pltpu.VMEM((_NBUF, _CHUNK, DIM), jnp.float32),
            pltpu.SemaphoreType.DMA,
            [pltpu.SemaphoreType.DMA] * _NBUF,
            [pltpu.SemaphoreType.DMA] * _NBUF,
        ],
    )


def _log1p(x):
    u = 1.0 + x
    d = u - 1.0
    return jnp.where(d == 0.0, x, jnp.log(u) * (x / d))


def _logaddexp(a, b):
    mx = jnp.maximum(a, b)
    return mx + _log1p(jnp.exp(-jnp.abs(a - b)))


def _softplus(x):
    return jnp.maximum(x, 0.0) + _log1p(jnp.exp(-jnp.abs(x)))


def _log_volume(bmin, bmax):
    return jnp.sum(jnp.log(_softplus((bmax - bmin) / BETA) * BETA + EPS),
                   axis=1, keepdims=True)


def _pred(h_min, h_max, t_min, t_max):
    meet_min = BETA * _logaddexp(h_min / BETA, t_min / BETA)
    meet_max = -BETA * _logaddexp(-h_max / BETA, -t_max / BETA)
    log_int = _log_volume(meet_min, meet_max)
    log_tail = _log_volume(t_min, t_max)
    return jnp.exp(jnp.minimum(log_int - log_tail, 0.0))


def _rownorm(x):
    return jnp.sqrt(jnp.sum(x * x, axis=1, keepdims=True))


_BB = 512


def _tc_loss_body(nb, mh, dh, mt, dt, mnh, dnh, mnt, dnt,
                  rth, rsh, rtt, rst, conf, out_ref, acc_ref):
    i = pl.program_id(0)

    @pl.when(i == 0)
    def _():
        acc_ref[0] = 0.0

    sc_h = jnp.exp(rsh[...])
    sc_t = jnp.exp(rst[...])
    edh = jnp.exp(dh[...])
    edt = jnp.exp(dt[...])

    h_min = mh[...] * sc_h + rth[...]
    h_max = h_min + edh * sc_h
    t_min = mt[...] * sc_t + rtt[...]
    t_max = t_min + edt * sc_t
    pos = _pred(h_min, h_max, t_min, t_max)

    nh_min = mnh[...] * sc_h + rth[...]
    nh_max = nh_min + jnp.exp(dnh[...]) * sc_h
    nt_min = mnt[...] * sc_t + rtt[...]
    nt_max = nt_min + jnp.exp(dnt[...]) * sc_t
    neg = _pred(nh_min, nh_max, nt_min, nt_max)

    se = (pos - conf[...]) ** 2 + neg * neg
    reg = (REG_DELTA * (_rownorm(edh) + _rownorm(edt))
           + REG_MIN * (_rownorm(mh[...]) + _rownorm(mt[...]))
           + REG_REL * (_rownorm(jnp.exp(rth[...])) + _rownorm(jnp.exp(rtt[...])))
           + REG_REL * (_rownorm(sc_h) + _rownorm(sc_t)))
    acc_ref[0] += jnp.sum(se) + jnp.sum(reg)

    @pl.when(i == nb - 1)
    def _():
        out_ref[...] = jnp.full((1, 1), acc_ref[0], jnp.float32)


def _make_tc_loss(nrows):
    nb = nrows // _BB
    row_spec = pl.BlockSpec((_BB, DIM), lambda i: (i, 0))
    conf_spec = pl.BlockSpec((_BB, 1), lambda i: (i, 0))
    return pl.pallas_call(
        functools.partial(_tc_loss_body, nb),
        grid=(nb,),
        in_specs=[row_spec] * 12 + [conf_spec],
        out_specs=pl.BlockSpec((1, 1), lambda i: (0, 0)),
        out_shape=jax.ShapeDtypeStruct((1, 1), jnp.float32),
        scratch_shapes=[pltpu.SMEM((1,), jnp.float32)],
    )


def kernel(ids, negative_samples, confidence, min_embedding, delta_embedding,
           rel_trans_for_head, rel_scale_for_head, rel_trans_for_tail,
           rel_scale_for_tail):
    ids = ids.astype(jnp.int32)
    neg = negative_samples.astype(jnp.int32)
    h = ids[:, 0]
    r = ids[:, 1]
    t = ids[:, 2]
    nh = neg[:, 0]
    nt = neg[:, 2]
    n = B // _NSPLIT
    sc_fn = _make_sc_gather(n)
    tc_fn = _make_tc_loss(n)
    conf2d = confidence.reshape(B, 1)
    partials = []
    for s in range(_NSPLIT):
        sl = slice(s * n, (s + 1) * n)
        gathered = sc_fn(min_embedding, delta_embedding,
                         rel_trans_for_head, rel_scale_for_head,
                         rel_trans_for_tail, rel_scale_for_tail,
                         h[sl], t[sl], nh[sl], nt[sl], r[sl])
        partials.append(tc_fn(*gathered, conf2d[sl]))
    total = partials[0]
    for p in partials[1:]:
        total = total + p
    return (total * (1.0 / B)).reshape(())
